# initial kernel scaffold (unmeasured)
import functools

import jax
import jax.numpy as jnp
from jax import lax
from jax.experimental import pallas as pl
from jax.experimental.pallas import tpu as pltpu

N_DEV = 4


def kernel(x, w_mat):
    m_global, k_per = x.shape
    _, n = w_mat.shape
    m_per = m_global // N_DEV

    def body(
        x_ref,
        w_ref,
        out_ref,
        comm_ref,
        amax_src_ref,
        amax_comm_ref,
        send_sems,
        recv_sems,
        credit_sem,
        amax_send_sems,
        amax_recv_sems,
    ):
        my = lax.axis_index("i")
        left = lax.rem(my + N_DEV - 1, N_DEV)
        right = lax.rem(my + 1, N_DEV)

        barrier_sem = pltpu.get_barrier_semaphore()
        for nbr in (left, right):
            pl.semaphore_signal(
                barrier_sem,
                inc=1,
                device_id=(nbr,),
                device_id_type=pl.DeviceIdType.MESH,
            )
        pl.semaphore_wait(barrier_sem, 2)

        def partial_chunk(owner):
            xc = x_ref[pl.ds(owner * m_per, m_per), :]
            return jnp.dot(
                xc,
                w_ref[...],
                preferred_element_type=jnp.float32,
                precision=lax.Precision.HIGHEST,
            )

        out_ref[...] = partial_chunk(lax.rem(my + 3, N_DEV))
        rdma0 = pltpu.make_async_remote_copy(
            src_ref=out_ref,
            dst_ref=comm_ref.at[0],
            send_sem=send_sems.at[0],
            recv_sem=recv_sems.at[0],
            device_id=(right,),
            device_id_type=pl.DeviceIdType.MESH,
        )
        rdma0.start()
        rdma0.wait()

        comm_ref[0] = comm_ref[0] + partial_chunk(lax.rem(my + 2, N_DEV))
        rdma1 = pltpu.make_async_remote_copy(
            src_ref=comm_ref.at[0],
            dst_ref=comm_ref.at[1],
            send_sem=send_sems.at[1],
            recv_sem=recv_sems.at[1],
            device_id=(right,),
            device_id_type=pl.DeviceIdType.MESH,
        )
        rdma1.start()
        rdma1.wait()
        pl.semaphore_signal(
            credit_sem,
            inc=1,
            device_id=(left,),
            device_id_type=pl.DeviceIdType.MESH,
        )

        comm_ref[1] = comm_ref[1] + partial_chunk(lax.rem(my + 1, N_DEV))
        pl.semaphore_wait(credit_sem, 1)
        rdma2 = pltpu.make_async_remote_copy(
            src_ref=comm_ref.at[1],
            dst_ref=comm_ref.at[0],
            send_sem=send_sems.at[2],
            recv_sem=recv_sems.at[0],
            device_id=(right,),
            device_id_type=pl.DeviceIdType.MESH,
        )
        rdma2.start()
        rdma2.wait()

        y = comm_ref[0] + partial_chunk(my)
        y = jnp.maximum(y, 0.0)

        running = jnp.full((8, 128), jnp.max(y), jnp.float32)
        for a in range(N_DEV - 1):
            amax_src_ref[...] = running
            r = pltpu.make_async_remote_copy(
                src_ref=amax_src_ref,
                dst_ref=amax_comm_ref.at[a],
                send_sem=amax_send_sems.at[a],
                recv_sem=amax_recv_sems.at[a],
                device_id=(right,),
                device_id_type=pl.DeviceIdType.MESH,
            )
            r.start()
            r.wait()
            running = jnp.maximum(running, amax_comm_ref[a])
        gmax = jnp.max(running)

        scale = gmax / 448.0
        q = jnp.minimum(y / scale, 448.0).astype(jnp.float8_e4m3fn)
        out_ref[...] = q.astype(jnp.float32) * scale

        @functools.partial(
            pl.run_scoped, second_barrier=pltpu.SemaphoreType.REGULAR
        )
        def _(second_barrier):
            for nbr in (left, right):
                pl.semaphore_signal(
                    second_barrier,
                    inc=1,
                    device_id=(nbr,),
                    device_id_type=pl.DeviceIdType.MESH,
                )
            pl.semaphore_wait(second_barrier, 2)

    return pl.pallas_call(
        body,
        out_shape=jax.ShapeDtypeStruct((m_per, n), jnp.float32),
        in_specs=[
            pl.BlockSpec(memory_space=pltpu.VMEM),
            pl.BlockSpec(memory_space=pltpu.VMEM),
        ],
        out_specs=pl.BlockSpec(memory_space=pltpu.VMEM),
        scratch_shapes=[
            pltpu.VMEM((2, m_per, n), jnp.float32),
            pltpu.VMEM((8, 128), jnp.float32),
            pltpu.VMEM((3, 8, 128), jnp.float32),
            pltpu.SemaphoreType.DMA((3,)),
            pltpu.SemaphoreType.DMA((2,)),
            pltpu.SemaphoreType.REGULAR,
            pltpu.SemaphoreType.DMA((3,)),
            pltpu.SemaphoreType.DMA((3,)),
        ],
        compiler_params=pltpu.CompilerParams(collective_id=0),
    )(x, w_mat)


# baseline (device time: 415580 ns/iter reference)
import functools

import jax
import jax.numpy as jnp
from jax import lax
from jax.experimental import pallas as pl
from jax.experimental.pallas import tpu as pltpu

N_DEV = 4
N_SPLIT = 4


def kernel(x, w_mat):
    m_global, k_per = x.shape
    _, n = w_mat.shape
    m_per = m_global // N_DEV
    n_blk = n // N_SPLIT

    def body(
        x_hbm,
        w_ref,
        out_ref,
        comm_ref,
        xc_ref,
        amax_src_ref,
        amax_comm_ref,
        send_sems,
        recv_sems,
        credit_sem,
        xdma_sem,
        amax_send_sems,
        amax_recv_sems,
    ):
        my = lax.axis_index("i")
        left = lax.rem(my + N_DEV - 1, N_DEV)
        right = lax.rem(my + 1, N_DEV)

        barrier_sem = pltpu.get_barrier_semaphore()
        for nbr in (left, right):
            pl.semaphore_signal(
                barrier_sem,
                inc=1,
                device_id=(nbr,),
                device_id_type=pl.DeviceIdType.MESH,
            )
        pl.semaphore_wait(barrier_sem, 2)

        def load_chunk(owner):
            cp = pltpu.make_async_copy(
                x_hbm.at[pl.ds(owner * m_per, m_per), :],
                xc_ref,
                xdma_sem,
            )
            cp.start()
            cp.wait()

        def dot_blk(b):
            return jnp.dot(
                xc_ref[...],
                w_ref[:, pl.ds(b * n_blk, n_blk)],
                preferred_element_type=jnp.float32,
                precision=lax.Precision.HIGHEST,
            )

        def accum_into(dst):
            for b in range(N_SPLIT):
                cols = pl.ds(b * n_blk, n_blk)
                dst[:, cols] = dst[:, cols] + dot_blk(b)

        load_chunk(lax.rem(my + 3, N_DEV))
        for b in range(N_SPLIT):
            out_ref[:, pl.ds(b * n_blk, n_blk)] = dot_blk(b)
        rdma0 = pltpu.make_async_remote_copy(
            src_ref=out_ref,
            dst_ref=comm_ref.at[0],
            send_sem=send_sems.at[0],
            recv_sem=recv_sems.at[0],
            device_id=(right,),
            device_id_type=pl.DeviceIdType.MESH,
        )
        rdma0.start()
        rdma0.wait()

        load_chunk(lax.rem(my + 2, N_DEV))
        accum_into(comm_ref.at[0])
        rdma1 = pltpu.make_async_remote_copy(
            src_ref=comm_ref.at[0],
            dst_ref=comm_ref.at[1],
            send_sem=send_sems.at[1],
            recv_sem=recv_sems.at[1],
            device_id=(right,),
            device_id_type=pl.DeviceIdType.MESH,
        )
        rdma1.start()
        rdma1.wait()
        pl.semaphore_signal(
            credit_sem,
            inc=1,
            device_id=(left,),
            device_id_type=pl.DeviceIdType.MESH,
        )

        load_chunk(lax.rem(my + 1, N_DEV))
        accum_into(comm_ref.at[1])
        pl.semaphore_wait(credit_sem, 1)
        rdma2 = pltpu.make_async_remote_copy(
            src_ref=comm_ref.at[1],
            dst_ref=comm_ref.at[0],
            send_sem=send_sems.at[2],
            recv_sem=recv_sems.at[0],
            device_id=(right,),
            device_id_type=pl.DeviceIdType.MESH,
        )
        rdma2.start()
        rdma2.wait()

        load_chunk(my)
        local_max = jnp.float32(0.0)
        for b in range(N_SPLIT):
            cols = pl.ds(b * n_blk, n_blk)
            yb = jnp.maximum(comm_ref[0, :, cols] + dot_blk(b), 0.0)
            out_ref[:, cols] = yb
            local_max = jnp.maximum(local_max, jnp.max(yb))

        running = jnp.full((8, 128), local_max, jnp.float32)
        for a in range(N_DEV - 1):
            amax_src_ref[...] = running
            r = pltpu.make_async_remote_copy(
                src_ref=amax_src_ref,
                dst_ref=amax_comm_ref.at[a],
                send_sem=amax_send_sems.at[a],
                recv_sem=amax_recv_sems.at[a],
                device_id=(right,),
                device_id_type=pl.DeviceIdType.MESH,
            )
            r.start()
            r.wait()
            running = jnp.maximum(running, amax_comm_ref[a])
        gmax = jnp.max(running)

        scale = gmax / 448.0
        inv_scale = 448.0 / gmax
        for b in range(N_SPLIT):
            cols = pl.ds(b * n_blk, n_blk)
            q = jnp.minimum(out_ref[:, cols] * inv_scale, 448.0).astype(
                jnp.float8_e4m3fn
            )
            out_ref[:, cols] = q.astype(jnp.float32) * scale

        @functools.partial(
            pl.run_scoped, second_barrier=pltpu.SemaphoreType.REGULAR
        )
        def _(second_barrier):
            for nbr in (left, right):
                pl.semaphore_signal(
                    second_barrier,
                    inc=1,
                    device_id=(nbr,),
                    device_id_type=pl.DeviceIdType.MESH,
                )
            pl.semaphore_wait(second_barrier, 2)

    return pl.pallas_call(
        body,
        out_shape=jax.ShapeDtypeStruct((m_per, n), jnp.float32),
        in_specs=[
            pl.BlockSpec(memory_space=pl.ANY),
            pl.BlockSpec(memory_space=pltpu.VMEM),
        ],
        out_specs=pl.BlockSpec(memory_space=pltpu.VMEM),
        scratch_shapes=[
            pltpu.VMEM((2, m_per, n), jnp.float32),
            pltpu.VMEM((m_per, k_per), jnp.float32),
            pltpu.VMEM((8, 128), jnp.float32),
            pltpu.VMEM((3, 8, 128), jnp.float32),
            pltpu.SemaphoreType.DMA((3,)),
            pltpu.SemaphoreType.DMA((2,)),
            pltpu.SemaphoreType.REGULAR,
            pltpu.SemaphoreType.DMA,
            pltpu.SemaphoreType.DMA((3,)),
            pltpu.SemaphoreType.DMA((3,)),
        ],
        compiler_params=pltpu.CompilerParams(
            collective_id=0,
            vmem_limit_bytes=44 * 1024 * 1024,
        ),
    )(x, w_mat)


# device time: 191589 ns/iter; 2.1691x vs baseline; 2.1691x over previous
import functools

import jax
import jax.numpy as jnp
from jax import lax
from jax.experimental import pallas as pl
from jax.experimental.pallas import tpu as pltpu

N_DEV = 4
NBLK = 512


def kernel(x, w_mat):
    m_global, k_per = x.shape
    _, n = w_mat.shape
    m_per = m_global // N_DEV
    h = m_per // 2
    nb = n // NBLK

    def body(
        x_hbm,
        w_ref,
        out_ref,
        comm_r,
        comm_l,
        stage_r,
        stage_l,
        xc_ref,
        amax_src_ref,
        amax_comm_ref,
        send_r,
        recv_r,
        send_l,
        recv_l,
        credit_r,
        credit_l,
        xdma_sems,
        amax_send_sems,
        amax_recv_sems,
    ):
        my = lax.axis_index("i")
        left = lax.rem(my + N_DEV - 1, N_DEV)
        right = lax.rem(my + 1, N_DEV)

        barrier_sem = pltpu.get_barrier_semaphore()
        for nbr in (left, right):
            pl.semaphore_signal(
                barrier_sem,
                inc=1,
                device_id=(nbr,),
                device_id_type=pl.DeviceIdType.MESH,
            )
        pl.semaphore_wait(barrier_sem, 2)

        TOP, BOT = 0, 1

        def load(slot, owner, half):
            cp = pltpu.make_async_copy(
                x_hbm.at[pl.ds(owner * m_per + half * h, h), :],
                xc_ref.at[slot],
                xdma_sems.at[slot],
            )
            cp.start()
            return cp

        def mm(dst, slot):
            for b in range(nb):
                cols = pl.ds(b * NBLK, NBLK)
                dst[:, cols] = jnp.dot(
                    xc_ref[slot],
                    w_ref[:, cols],
                    preferred_element_type=jnp.float32,
                    precision=lax.Precision.HIGHEST,
                )

        def acc(dst, src):
            for b in range(nb):
                cols = pl.ds(b * NBLK, NBLK)
                dst[:, cols] = dst[:, cols] + src[:, cols]

        def rdma(src, dst, ssem, rsem, dev):
            return pltpu.make_async_remote_copy(
                src_ref=src,
                dst_ref=dst,
                send_sem=ssem,
                recv_sem=rsem,
                device_id=(dev,),
                device_id_type=pl.DeviceIdType.MESH,
            )

        out_top = out_ref.at[pl.ds(0, h)]
        out_bot = out_ref.at[pl.ds(h, h)]

        c0 = load(0, lax.rem(my + 3, N_DEV), TOP)
        c1 = load(1, lax.rem(my + 1, N_DEV), BOT)
        c0.wait()
        mm(stage_r, 0)
        c1.wait()
        mm(stage_l, 1)
        r0 = rdma(stage_r, comm_r.at[0], send_r.at[0], recv_r.at[0], right)
        l0 = rdma(stage_l, comm_l.at[0], send_l.at[0], recv_l.at[0], left)
        r0.start()
        l0.start()

        c2 = load(2, lax.rem(my + 2, N_DEV), TOP)
        c3 = load(3, lax.rem(my + 2, N_DEV), BOT)
        c2.wait()
        mm(out_top, 2)
        c3.wait()
        mm(out_bot, 3)

        r0.wait()
        l0.wait()
        acc(out_top, comm_r.at[0])
        acc(out_bot, comm_l.at[0])
        pl.semaphore_signal(
            credit_r, inc=1, device_id=(left,),
            device_id_type=pl.DeviceIdType.MESH,
        )
        pl.semaphore_signal(
            credit_l, inc=1, device_id=(right,),
            device_id_type=pl.DeviceIdType.MESH,
        )
        r1 = rdma(out_top, comm_r.at[1], send_r.at[1], recv_r.at[1], right)
        l1 = rdma(out_bot, comm_l.at[1], send_l.at[1], recv_l.at[1], left)
        r1.start()
        l1.start()

        c0 = load(0, lax.rem(my + 1, N_DEV), TOP)
        c1 = load(1, lax.rem(my + 3, N_DEV), BOT)
        c0.wait()
        mm(stage_r, 0)
        c1.wait()
        mm(stage_l, 1)

        r1.wait()
        l1.wait()
        acc(stage_r, comm_r.at[1])
        acc(stage_l, comm_l.at[1])
        pl.semaphore_wait(credit_r, 1)
        pl.semaphore_wait(credit_l, 1)
        r2 = rdma(stage_r, comm_r.at[0], send_r.at[2], recv_r.at[0], right)
        l2 = rdma(stage_l, comm_l.at[0], send_l.at[2], recv_l.at[0], left)
        r2.start()
        l2.start()

        c2 = load(2, my, TOP)
        c3 = load(3, my, BOT)
        c2.wait()
        mm(out_top, 2)
        c3.wait()
        mm(out_bot, 3)

        r2.wait()
        l2.wait()

        local_max = jnp.float32(0.0)
        for half_ref, comm in ((out_top, comm_r), (out_bot, comm_l)):
            for b in range(nb):
                cols = pl.ds(b * NBLK, NBLK)
                yb = jnp.maximum(half_ref[:, cols] + comm[0, :, cols], 0.0)
                half_ref[:, cols] = yb
                local_max = jnp.maximum(local_max, jnp.max(yb))

        running = jnp.full((8, 128), local_max, jnp.float32)
        for a in range(N_DEV - 1):
            amax_src_ref[...] = running
            r = rdma(
                amax_src_ref,
                amax_comm_ref.at[a],
                amax_send_sems.at[a],
                amax_recv_sems.at[a],
                right,
            )
            r.start()
            r.wait()
            running = jnp.maximum(running, amax_comm_ref[a])
        gmax = jnp.max(running)

        scale = gmax / 448.0
        inv_scale = 448.0 / gmax
        for b in range(nb):
            cols = pl.ds(b * NBLK, NBLK)
            q = jnp.minimum(out_ref[:, cols] * inv_scale, 448.0).astype(
                jnp.float8_e4m3fn
            )
            out_ref[:, cols] = q.astype(jnp.float32) * scale

        @functools.partial(
            pl.run_scoped, second_barrier=pltpu.SemaphoreType.REGULAR
        )
        def _(second_barrier):
            for nbr in (left, right):
                pl.semaphore_signal(
                    second_barrier,
                    inc=1,
                    device_id=(nbr,),
                    device_id_type=pl.DeviceIdType.MESH,
                )
            pl.semaphore_wait(second_barrier, 2)

    return pl.pallas_call(
        body,
        out_shape=jax.ShapeDtypeStruct((m_per, n), jnp.float32),
        in_specs=[
            pl.BlockSpec(memory_space=pl.ANY),
            pl.BlockSpec(memory_space=pltpu.VMEM),
        ],
        out_specs=pl.BlockSpec(memory_space=pltpu.VMEM),
        scratch_shapes=[
            pltpu.VMEM((2, h, n), jnp.float32),
            pltpu.VMEM((2, h, n), jnp.float32),
            pltpu.VMEM((h, n), jnp.float32),
            pltpu.VMEM((h, n), jnp.float32),
            pltpu.VMEM((4, h, k_per), jnp.float32),
            pltpu.VMEM((8, 128), jnp.float32),
            pltpu.VMEM((3, 8, 128), jnp.float32),
            pltpu.SemaphoreType.DMA((3,)),
            pltpu.SemaphoreType.DMA((2,)),
            pltpu.SemaphoreType.DMA((3,)),
            pltpu.SemaphoreType.DMA((2,)),
            pltpu.SemaphoreType.REGULAR,
            pltpu.SemaphoreType.REGULAR,
            pltpu.SemaphoreType.DMA((4,)),
            pltpu.SemaphoreType.DMA((3,)),
            pltpu.SemaphoreType.DMA((3,)),
        ],
        compiler_params=pltpu.CompilerParams(
            collective_id=0,
            vmem_limit_bytes=44 * 1024 * 1024,
        ),
    )(x, w_mat)


# device time: 164408 ns/iter; 2.5277x vs baseline; 1.1653x over previous
import functools

import jax
import jax.numpy as jnp
from jax import lax
from jax.experimental import pallas as pl
from jax.experimental.pallas import tpu as pltpu

N_DEV = 4
NB = 4


def kernel(x, w_mat):
    m_global, k_per = x.shape
    _, n = w_mat.shape
    m_per = m_global // N_DEV
    h = m_per // 2
    nblk = n // NB

    def body(
        x_hbm,
        w_hbm,
        out_ref,
        comm_r,
        comm_l,
        stage_ra,
        stage_la,
        stage_rb,
        stage_lb,
        xc_hi,
        xc_lo,
        xland,
        w_hi,
        w_lo,
        wland,
        amax_src_ref,
        amax_comm_ref,
        send_r,
        recv_r,
        send_l,
        recv_l,
        credit_r,
        credit_l,
        xdma_sem,
        wdma_sem,
        amax_send_sems,
        amax_recv_sems,
    ):
        my = lax.axis_index("i")
        left = lax.rem(my + N_DEV - 1, N_DEV)
        right = lax.rem(my + 1, N_DEV)

        TOP, BOT = 0, 1

        def load(owner, half):
            cp = pltpu.make_async_copy(
                x_hbm.at[pl.ds(owner * m_per + half * h, h), :],
                xland,
                xdma_sem,
            )
            cp.start()
            return cp

        def xconv(slot):
            f = xland[...]
            hi = f.astype(jnp.bfloat16)
            xc_hi[slot] = hi
            xc_lo[slot] = (f - hi.astype(jnp.float32)).astype(jnp.bfloat16)

        def load2(owner_top, owner_bot):
            load(owner_top, TOP).wait()
            xconv(0)
            load(owner_bot, BOT).wait()
            xconv(1)

        def wconv(b):
            cols = pl.ds(b * nblk, nblk)
            for piece in range(2):
                rows = pl.ds(piece * (k_per // 2), k_per // 2)
                cp = pltpu.make_async_copy(
                    w_hbm.at[rows, cols], wland, wdma_sem
                )
                cp.start()
                cp.wait()
                f = wland[...]
                hi = f.astype(jnp.bfloat16)
                w_hi[rows, cols] = hi
                w_lo[rows, cols] = (
                    f - hi.astype(jnp.float32)
                ).astype(jnp.bfloat16)

        c0 = load(lax.rem(my + 3, N_DEV), TOP)

        barrier_sem = pltpu.get_barrier_semaphore()
        for nbr in (left, right):
            pl.semaphore_signal(
                barrier_sem,
                inc=1,
                device_id=(nbr,),
                device_id_type=pl.DeviceIdType.MESH,
            )
        pl.semaphore_wait(barrier_sem, 2)

        def mmb(dst, slot, b):
            cols = pl.ds(b * nblk, nblk)
            xh, xl = xc_hi[slot], xc_lo[slot]
            wh, wl = w_hi[:, cols], w_lo[:, cols]
            dst[...] = jnp.dot(xh, wh, preferred_element_type=jnp.float32)
            dst[...] = dst[...] + jnp.dot(
                xh, wl, preferred_element_type=jnp.float32
            )
            dst[...] = dst[...] + jnp.dot(
                xl, wh, preferred_element_type=jnp.float32
            )

        def rdma(src, dst, ssem, rsem, dev):
            return pltpu.make_async_remote_copy(
                src_ref=src,
                dst_ref=dst,
                send_sem=ssem,
                recv_sem=rsem,
                device_id=(dev,),
                device_id_type=pl.DeviceIdType.MESH,
            )

        c0.wait()
        xconv(0)
        load(lax.rem(my + 1, N_DEV), BOT).wait()
        xconv(1)
        r0, l0 = [], []
        for b in range(NB):
            wconv(b)
            mmb(stage_ra.at[b], 0, b)
            rb = rdma(
                stage_ra.at[b], comm_r.at[0, b],
                send_r.at[0, b], recv_r.at[0, b], right,
            )
            rb.start()
            r0.append(rb)
            mmb(stage_la.at[b], 1, b)
            lb = rdma(
                stage_la.at[b], comm_l.at[0, b],
                send_l.at[0, b], recv_l.at[0, b], left,
            )
            lb.start()
            l0.append(lb)

        load2(lax.rem(my + 2, N_DEV), lax.rem(my + 2, N_DEV))
        for b in range(NB):
            mmb(stage_rb.at[b], 0, b)
            mmb(stage_lb.at[b], 1, b)

        r1, l1 = [], []
        for b in range(NB):
            r0[b].wait_recv()
            stage_rb[b] = stage_rb[b] + comm_r[0, b]
            rb = rdma(
                stage_rb.at[b], comm_r.at[1, b],
                send_r.at[1, b], recv_r.at[1, b], right,
            )
            rb.start()
            r1.append(rb)
            l0[b].wait_recv()
            stage_lb[b] = stage_lb[b] + comm_l[0, b]
            lb = rdma(
                stage_lb.at[b], comm_l.at[1, b],
                send_l.at[1, b], recv_l.at[1, b], left,
            )
            lb.start()
            l1.append(lb)
        pl.semaphore_signal(
            credit_r, inc=1, device_id=(left,),
            device_id_type=pl.DeviceIdType.MESH,
        )
        pl.semaphore_signal(
            credit_l, inc=1, device_id=(right,),
            device_id_type=pl.DeviceIdType.MESH,
        )

        for b in range(NB):
            r0[b].wait_send()
            l0[b].wait_send()
        load2(lax.rem(my + 1, N_DEV), lax.rem(my + 3, N_DEV))
        for b in range(NB):
            mmb(stage_ra.at[b], 0, b)
            mmb(stage_la.at[b], 1, b)

        pl.semaphore_wait(credit_r, 1)
        pl.semaphore_wait(credit_l, 1)
        r2, l2 = [], []
        for b in range(NB):
            r1[b].wait_recv()
            stage_ra[b] = stage_ra[b] + comm_r[1, b]
            rb = rdma(
                stage_ra.at[b], comm_r.at[0, b],
                send_r.at[2, b], recv_r.at[0, b], right,
            )
            rb.start()
            r2.append(rb)
            l1[b].wait_recv()
            stage_la[b] = stage_la[b] + comm_l[1, b]
            lb = rdma(
                stage_la.at[b], comm_l.at[0, b],
                send_l.at[2, b], recv_l.at[0, b], left,
            )
            lb.start()
            l2.append(lb)

        load2(my, my)
        out_top = out_ref.at[pl.ds(0, h)]
        out_bot = out_ref.at[pl.ds(h, h)]
        for b in range(NB):
            cols = pl.ds(b * nblk, nblk)
            mmb(out_top.at[:, cols], 0, b)
            mmb(out_bot.at[:, cols], 1, b)

        local_max = jnp.float32(0.0)
        for b in range(NB):
            cols = pl.ds(b * nblk, nblk)
            r2[b].wait_recv()
            yt = jnp.maximum(out_top[:, cols] + comm_r[0, b], 0.0)
            out_top[:, cols] = yt
            local_max = jnp.maximum(local_max, jnp.max(yt))
            l2[b].wait_recv()
            yb = jnp.maximum(out_bot[:, cols] + comm_l[0, b], 0.0)
            out_bot[:, cols] = yb
            local_max = jnp.maximum(local_max, jnp.max(yb))

        amax_src_ref[...] = jnp.full((8, 128), local_max, jnp.float32)
        amax_comm_ref[my] = amax_src_ref[...]
        peers = [lax.rem(my + d, N_DEV) for d in (1, 2, 3)]
        sends = []
        for j, q in enumerate(peers):
            s = rdma(
                amax_src_ref,
                amax_comm_ref.at[my],
                amax_send_sems.at[j],
                amax_recv_sems.at[my],
                q,
            )
            s.start()
            sends.append(s)
        for q in peers:
            rdma(
                amax_src_ref,
                amax_comm_ref.at[q],
                amax_send_sems.at[0],
                amax_recv_sems.at[q],
                right,
            ).wait_recv()
        gmax = jnp.max(amax_comm_ref[...])

        scale = gmax / 448.0
        inv_scale = 448.0 / gmax
        for b in range(NB):
            cols = pl.ds(b * nblk, nblk)
            q8 = jnp.minimum(out_ref[:, cols] * inv_scale, 448.0).astype(
                jnp.float8_e4m3fn
            )
            out_ref[:, cols] = q8.astype(jnp.float32) * scale

        for d in sends:
            d.wait_send()
        for b in range(NB):
            r1[b].wait_send()
            l1[b].wait_send()
            r2[b].wait_send()
            l2[b].wait_send()

        @functools.partial(
            pl.run_scoped, second_barrier=pltpu.SemaphoreType.REGULAR
        )
        def _(second_barrier):
            for nbr in (left, right):
                pl.semaphore_signal(
                    second_barrier,
                    inc=1,
                    device_id=(nbr,),
                    device_id_type=pl.DeviceIdType.MESH,
                )
            pl.semaphore_wait(second_barrier, 2)

    return pl.pallas_call(
        body,
        out_shape=jax.ShapeDtypeStruct((m_per, n), jnp.float32),
        in_specs=[
            pl.BlockSpec(memory_space=pl.ANY),
            pl.BlockSpec(memory_space=pl.ANY),
        ],
        out_specs=pl.BlockSpec(memory_space=pltpu.VMEM),
        scratch_shapes=[
            pltpu.VMEM((2, NB, h, n // NB), jnp.float32),
            pltpu.VMEM((2, NB, h, n // NB), jnp.float32),
            pltpu.VMEM((NB, h, n // NB), jnp.float32),
            pltpu.VMEM((NB, h, n // NB), jnp.float32),
            pltpu.VMEM((NB, h, n // NB), jnp.float32),
            pltpu.VMEM((NB, h, n // NB), jnp.float32),
            pltpu.VMEM((2, h, k_per), jnp.bfloat16),
            pltpu.VMEM((2, h, k_per), jnp.bfloat16),
            pltpu.VMEM((h, k_per), jnp.float32),
            pltpu.VMEM((k_per, n), jnp.bfloat16),
            pltpu.VMEM((k_per, n), jnp.bfloat16),
            pltpu.VMEM((k_per // 2, n // NB), jnp.float32),
            pltpu.VMEM((8, 128), jnp.float32),
            pltpu.VMEM((N_DEV, 8, 128), jnp.float32),
            pltpu.SemaphoreType.DMA((3, NB)),
            pltpu.SemaphoreType.DMA((2, NB)),
            pltpu.SemaphoreType.DMA((3, NB)),
            pltpu.SemaphoreType.DMA((2, NB)),
            pltpu.SemaphoreType.REGULAR,
            pltpu.SemaphoreType.REGULAR,
            pltpu.SemaphoreType.DMA,
            pltpu.SemaphoreType.DMA,
            pltpu.SemaphoreType.DMA((3,)),
            pltpu.SemaphoreType.DMA((N_DEV,)),
        ],
        compiler_params=pltpu.CompilerParams(
            collective_id=0,
            vmem_limit_bytes=55 * 1024 * 1024 + 512 * 1024,
        ),
    )(x, w_mat)
